# baseline (speedup 1.0000x reference)
import functools

import jax
import jax.numpy as jnp
from jax import lax
from jax.experimental import pallas as pl
from jax.experimental.pallas import tpu as pltpu
from jax.experimental.pallas import tpu_sc as plsc

_NTILES = 32
_CE = 64
_CQ = 64


def _segment_sum_partials(x, epack, ew):
    n, d = x.shape
    nchunks = epack.shape[0]
    iters = -(-nchunks // _NTILES)
    zrows = 40
    ngroups = n // zrows
    nbuf = 5
    mesh = plsc.VectorSubcoreMesh(core_axis_name="c", subcore_axis_name="s")

    @functools.partial(
        pl.kernel,
        mesh=mesh,
        out_type=jax.ShapeDtypeStruct((2, n, d), jnp.float32),
        scratch_types=[
            pltpu.VMEM((nbuf, 2, _CE), jnp.int32),
            pltpu.VMEM((nbuf, _CE), jnp.float32),
            pltpu.VMEM((nbuf, _CE, d), jnp.float32),
            pltpu.VMEM_SHARED((n, d), jnp.float32),
            pltpu.SemaphoreType.DMA,
            pltpu.SemaphoreType.DMA,
            pltpu.SemaphoreType.DMA,
            pltpu.SemaphoreType.DMA,
        ],
    )
    def k(x_hbm, epack_hbm, ew_hbm, part_hbm, idxw_v, w_v, rows_v,
          shared, sem_i, sem_w, sem_g, sem_s):
        c = lax.axis_index("c")
        s = lax.axis_index("s")
        wid = s * 2 + c

        zvec = jnp.zeros((16,), jnp.float32)
        for r in range(zrows):
            for db in range(d // 16):
                rows_v[0, r, pl.ds(db * 16, 16)] = zvec

        def zero_body(i, carry):
            g = i * 16 + s

            @pl.when(g < ngroups)
            def _():
                pltpu.sync_copy(rows_v.at[0, pl.ds(0, zrows)],
                                shared.at[pl.ds(g * zrows, zrows)])

            return carry

        lax.fori_loop(0, -(-ngroups // 16), zero_body, 0)
        plsc.subcore_barrier()

        for m in range(nbuf - 1):
            pltpu.sync_copy(epack_hbm.at[wid + m * _NTILES], idxw_v.at[m])
            pltpu.sync_copy(ew_hbm.at[wid + m * _NTILES], w_v.at[m])
            pltpu.async_copy(x_hbm.at[idxw_v.at[m, 0]], rows_v.at[m], sem_g)

        def scale_rows(b):
            def scale_body(g, carry2):
                w16 = w_v[b, pl.ds(g * 16, 16)]
                for l in range(16):
                    w = w16[l]
                    ei = g * 16 + l
                    for db in range(d // 16):
                        sl = pl.ds(db * 16, 16)
                        rows_v[b, ei, sl] = rows_v[b, ei, sl] * w
                return carry2

            lax.fori_loop(0, _CE // 16, scale_body, 0)

        def outer(i, carry):
            for b in range(nbuf):
                kk = i * nbuf + b
                sn = (b + nbuf - 1) % nbuf
                j = kk * _NTILES + wid
                jnx = j + (nbuf - 1) * _NTILES
                jp1 = j - _NTILES

                @pl.when((kk >= 1) & (jp1 < nchunks))
                def _():
                    pltpu.make_async_copy(
                        rows_v.at[sn], shared.at[idxw_v.at[sn, 1]],
                        sem_s).wait()

                @pl.when(jnx < nchunks)
                def _():
                    pltpu.async_copy(epack_hbm.at[jnx], idxw_v.at[sn], sem_i)
                    pltpu.async_copy(ew_hbm.at[jnx], w_v.at[sn], sem_w)

                @pl.when(j < nchunks)
                def _():
                    pltpu.make_async_copy(
                        x_hbm.at[idxw_v.at[b, 0]], rows_v.at[b], sem_g).wait()
                    scale_rows(b)
                    pltpu.async_copy(rows_v.at[b], shared.at[idxw_v.at[b, 1]],
                                     sem_s, add=True)

                @pl.when(jnx < nchunks)
                def _():
                    pltpu.make_async_copy(
                        epack_hbm.at[0], idxw_v.at[sn], sem_i).wait()
                    pltpu.make_async_copy(
                        ew_hbm.at[0], w_v.at[sn], sem_w).wait()
                    pltpu.async_copy(
                        x_hbm.at[idxw_v.at[sn, 0]], rows_v.at[sn], sem_g)

            return carry

        lax.fori_loop(0, iters // nbuf + 2, outer, 0)
        plsc.subcore_barrier()

        def out_body(i, carry):
            g = i * 16 + s

            @pl.when(g < ngroups)
            def _():
                pltpu.sync_copy(shared.at[pl.ds(g * zrows, zrows)],
                                part_hbm.at[c, pl.ds(g * zrows, zrows)])

            return carry

        lax.fori_loop(0, -(-ngroups // 16), out_body, 0)

    return k(x, epack, ew)


def _linear(part0, part1, w):
    n, d = part0.shape
    blk = 400

    def mm(a_ref, b_ref, w_ref, o_ref):
        o_ref[...] = jnp.dot(a_ref[...] + b_ref[...], w_ref[...],
                             preferred_element_type=jnp.float32)

    return pl.pallas_call(
        mm,
        grid=(n // blk,),
        in_specs=[
            pl.BlockSpec((blk, d), lambda i: (i, 0)),
            pl.BlockSpec((blk, d), lambda i: (i, 0)),
            pl.BlockSpec((d, d), lambda i: (0, 0)),
        ],
        out_specs=pl.BlockSpec((blk, d), lambda i: (i, 0)),
        out_shape=jax.ShapeDtypeStruct((n, d), jnp.float32),
    )(part0, part1, w)


def _pair_partial_dots(h, qpack):
    n, d = h.shape
    nchunks = qpack.shape[0]
    iters = -(-nchunks // _NTILES)
    nbuf = 4
    mesh = plsc.VectorSubcoreMesh(core_axis_name="c", subcore_axis_name="s")

    @functools.partial(
        pl.kernel,
        mesh=mesh,
        out_type=jax.ShapeDtypeStruct((nchunks * 8, 128), jnp.float32),
        scratch_types=[
            pltpu.VMEM((nbuf, 2 * _CQ), jnp.int32),
            pltpu.VMEM((nbuf, 2 * _CQ, d), jnp.float32),
            pltpu.VMEM((nbuf, 8, 128), jnp.float32),
            pltpu.SemaphoreType.DMA,
            pltpu.SemaphoreType.DMA,
            pltpu.SemaphoreType.DMA,
        ],
    )
    def k(h_hbm, qpack_hbm, out_hbm, pairs_v, rows_v, pbuf_v,
          sem_p, sem_g, sem_o):
        c = lax.axis_index("c")
        s = lax.axis_index("s")
        wid = s * 2 + c

        for m in range(nbuf - 1):
            pltpu.sync_copy(qpack_hbm.at[wid + m * _NTILES], pairs_v.at[m])
            pltpu.async_copy(h_hbm.at[pairs_v.at[m]], rows_v.at[m], sem_g)

        def compute_chunk(b):
            def dot_body(g, carry2):
                for l in range(16):
                    qi = g * 16 + l
                    acc = None
                    for db in range(d // 16):
                        sl = pl.ds(db * 16, 16)
                        prod = (rows_v[b, qi, sl] *
                                rows_v[b, _CQ + qi, sl])
                        acc = prod if acc is None else acc + prod
                    pbuf_v[b, g * 2 + l // 8, pl.ds((l % 8) * 16, 16)] = acc
                return carry2

            lax.fori_loop(0, _CQ // 16, dot_body, 0)

        def outer(i, carry):
            for b in range(nbuf):
                kk = i * nbuf + b
                s3 = (b + 3) % nbuf
                j = kk * _NTILES + wid
                jn3 = j + 3 * _NTILES
                jp4 = j - 4 * _NTILES

                @pl.when(jn3 < nchunks)
                def _():
                    pltpu.async_copy(qpack_hbm.at[jn3], pairs_v.at[s3], sem_p)

                @pl.when((kk >= 4) & (jp4 < nchunks))
                def _():
                    pltpu.make_async_copy(
                        pbuf_v.at[b], out_hbm.at[pl.ds(0, 8)], sem_o).wait()

                @pl.when(j < nchunks)
                def _():
                    pltpu.make_async_copy(
                        h_hbm.at[pairs_v.at[b]], rows_v.at[b], sem_g).wait()
                    compute_chunk(b)
                    pltpu.async_copy(
                        pbuf_v.at[b], out_hbm.at[pl.ds(j * 8, 8)], sem_o)

                @pl.when(jn3 < nchunks)
                def _():
                    pltpu.make_async_copy(
                        qpack_hbm.at[0], pairs_v.at[s3], sem_p).wait()
                    pltpu.async_copy(
                        h_hbm.at[pairs_v.at[s3]], rows_v.at[s3], sem_g)

            return carry

        lax.fori_loop(0, iters // nbuf + 2, outer, 0)

    return k(h, qpack)


def _lane_reduce(partials, q):
    q8 = partials.shape[0]
    blk = 1000

    def red(p_ref, o_ref):
        r = lax.broadcasted_iota(jnp.int32, (128, 8), 0) // 16
        t = lax.broadcasted_iota(jnp.int32, (128, 8), 1)
        mask = (r == t).astype(jnp.float32)
        o_ref[...] = jnp.dot(p_ref[...], mask,
                             preferred_element_type=jnp.float32)

    out = pl.pallas_call(
        red,
        grid=(q8 // blk,),
        in_specs=[pl.BlockSpec((blk, 128), lambda i: (i, 0))],
        out_specs=pl.BlockSpec((blk, 8), lambda i: (i, 0)),
        out_shape=jax.ShapeDtypeStruct((q8, 8), jnp.float32),
    )(partials)
    return out.reshape(q)


def kernel(x, edge_index, edge_weight, edges, W):
    e = edge_index.shape[1]
    q = edges.shape[1]
    epack = jnp.stack(
        [edge_index[0].reshape(e // _CE, _CE),
         edge_index[1].reshape(e // _CE, _CE)], axis=1)
    ew = edge_weight.reshape(e // _CE, _CE)
    qpack = jnp.concatenate(
        [edges[0].reshape(q // _CQ, _CQ),
         edges[1].reshape(q // _CQ, _CQ)], axis=1)

    part = _segment_sum_partials(x, epack, ew)
    h = _linear(part[0], part[1], W)
    partials = _pair_partial_dots(h, qpack)
    return _lane_reduce(partials, q)


# speedup vs baseline: 1.1326x; 1.1326x over previous
import functools

import jax
import jax.numpy as jnp
from jax import lax
from jax.experimental import pallas as pl
from jax.experimental.pallas import tpu as pltpu
from jax.experimental.pallas import tpu_sc as plsc

_NTILES = 32
_CE = 128
_CQ = 64


def _segment_sum_partials(x, epack, ew):
    n, d = x.shape
    nchunks = epack.shape[0]
    iters = -(-nchunks // _NTILES)
    zrows = 40
    ngroups = n // zrows
    nbuf = 3
    mesh = plsc.VectorSubcoreMesh(core_axis_name="c", subcore_axis_name="s")

    @functools.partial(
        pl.kernel,
        mesh=mesh,
        out_type=jax.ShapeDtypeStruct((2, n, d), jnp.float32),
        scratch_types=[
            pltpu.VMEM((nbuf, 2, _CE), jnp.int32),
            pltpu.VMEM((nbuf, _CE), jnp.float32),
            pltpu.VMEM((nbuf, _CE, d), jnp.float32),
            pltpu.VMEM_SHARED((n, d), jnp.float32),
            pltpu.SemaphoreType.DMA,
            pltpu.SemaphoreType.DMA,
            pltpu.SemaphoreType.DMA,
            pltpu.SemaphoreType.DMA,
        ],
    )
    def k(x_hbm, epack_hbm, ew_hbm, part_hbm, idxw_v, w_v, rows_v,
          shared, sem_i, sem_w, sem_g, sem_s):
        c = lax.axis_index("c")
        s = lax.axis_index("s")
        wid = s * 2 + c

        zvec = jnp.zeros((16,), jnp.float32)
        for r in range(zrows):
            for db in range(d // 16):
                rows_v[0, r, pl.ds(db * 16, 16)] = zvec

        def zero_body(i, carry):
            g = i * 16 + s

            @pl.when(g < ngroups)
            def _():
                pltpu.sync_copy(rows_v.at[0, pl.ds(0, zrows)],
                                shared.at[pl.ds(g * zrows, zrows)])

            return carry

        lax.fori_loop(0, -(-ngroups // 16), zero_body, 0)
        plsc.subcore_barrier()

        for m in range(nbuf - 1):
            pltpu.sync_copy(epack_hbm.at[wid + m * _NTILES], idxw_v.at[m])
            pltpu.sync_copy(ew_hbm.at[wid + m * _NTILES], w_v.at[m])
            pltpu.async_copy(x_hbm.at[idxw_v.at[m, 0]], rows_v.at[m], sem_g)

        def scale_rows(b):
            def scale_body(g, carry2):
                w16 = w_v[b, pl.ds(g * 16, 16)]
                for l in range(16):
                    w = w16[l]
                    ei = g * 16 + l
                    for db in range(d // 16):
                        sl = pl.ds(db * 16, 16)
                        rows_v[b, ei, sl] = rows_v[b, ei, sl] * w
                return carry2

            lax.fori_loop(0, _CE // 16, scale_body, 0)

        def outer(i, carry):
            for b in range(nbuf):
                kk = i * nbuf + b
                sn = (b + nbuf - 1) % nbuf
                j = kk * _NTILES + wid
                jnx = j + (nbuf - 1) * _NTILES
                jp1 = j - _NTILES

                @pl.when((kk >= 1) & (jp1 < nchunks))
                def _():
                    pltpu.make_async_copy(
                        rows_v.at[sn], shared.at[idxw_v.at[sn, 1]],
                        sem_s).wait()

                @pl.when(jnx < nchunks)
                def _():
                    pltpu.async_copy(epack_hbm.at[jnx], idxw_v.at[sn], sem_i)
                    pltpu.async_copy(ew_hbm.at[jnx], w_v.at[sn], sem_w)

                @pl.when(j < nchunks)
                def _():
                    pltpu.make_async_copy(
                        x_hbm.at[idxw_v.at[b, 0]], rows_v.at[b], sem_g).wait()
                    scale_rows(b)
                    pltpu.async_copy(rows_v.at[b], shared.at[idxw_v.at[b, 1]],
                                     sem_s, add=True)

                @pl.when(jnx < nchunks)
                def _():
                    pltpu.make_async_copy(
                        epack_hbm.at[0], idxw_v.at[sn], sem_i).wait()
                    pltpu.make_async_copy(
                        ew_hbm.at[0], w_v.at[sn], sem_w).wait()
                    pltpu.async_copy(
                        x_hbm.at[idxw_v.at[sn, 0]], rows_v.at[sn], sem_g)

            return carry

        lax.fori_loop(0, iters // nbuf + 2, outer, 0)
        plsc.subcore_barrier()

        def out_body(i, carry):
            g = i * 16 + s

            @pl.when(g < ngroups)
            def _():
                pltpu.sync_copy(shared.at[pl.ds(g * zrows, zrows)],
                                part_hbm.at[c, pl.ds(g * zrows, zrows)])

            return carry

        lax.fori_loop(0, -(-ngroups // 16), out_body, 0)

    return k(x, epack, ew)


def _linear(part0, part1, w):
    n, d = part0.shape
    blk = 400

    def mm(a_ref, b_ref, w_ref, o_ref):
        o_ref[...] = jnp.dot(a_ref[...] + b_ref[...], w_ref[...],
                             preferred_element_type=jnp.float32)

    return pl.pallas_call(
        mm,
        grid=(n // blk,),
        in_specs=[
            pl.BlockSpec((blk, d), lambda i: (i, 0)),
            pl.BlockSpec((blk, d), lambda i: (i, 0)),
            pl.BlockSpec((d, d), lambda i: (0, 0)),
        ],
        out_specs=pl.BlockSpec((blk, d), lambda i: (i, 0)),
        out_shape=jax.ShapeDtypeStruct((n, d), jnp.float32),
    )(part0, part1, w)


def _pair_partial_dots(h, qpack):
    n, d = h.shape
    nchunks = qpack.shape[0]
    iters = -(-nchunks // _NTILES)
    nbuf = 4
    mesh = plsc.VectorSubcoreMesh(core_axis_name="c", subcore_axis_name="s")

    @functools.partial(
        pl.kernel,
        mesh=mesh,
        out_type=jax.ShapeDtypeStruct((nchunks * 8, 128), jnp.float32),
        scratch_types=[
            pltpu.VMEM((nbuf, 2 * _CQ), jnp.int32),
            pltpu.VMEM((nbuf, 2 * _CQ, d), jnp.float32),
            pltpu.VMEM((nbuf, 8, 128), jnp.float32),
            pltpu.SemaphoreType.DMA,
            pltpu.SemaphoreType.DMA,
            pltpu.SemaphoreType.DMA,
        ],
    )
    def k(h_hbm, qpack_hbm, out_hbm, pairs_v, rows_v, pbuf_v,
          sem_p, sem_g, sem_o):
        c = lax.axis_index("c")
        s = lax.axis_index("s")
        wid = s * 2 + c

        for m in range(nbuf - 1):
            pltpu.sync_copy(qpack_hbm.at[wid + m * _NTILES], pairs_v.at[m])
            pltpu.async_copy(h_hbm.at[pairs_v.at[m]], rows_v.at[m], sem_g)

        def compute_chunk(b):
            def dot_body(g, carry2):
                for l in range(16):
                    qi = g * 16 + l
                    acc = None
                    for db in range(d // 16):
                        sl = pl.ds(db * 16, 16)
                        prod = (rows_v[b, qi, sl] *
                                rows_v[b, _CQ + qi, sl])
                        acc = prod if acc is None else acc + prod
                    pbuf_v[b, g * 2 + l // 8, pl.ds((l % 8) * 16, 16)] = acc
                return carry2

            lax.fori_loop(0, _CQ // 16, dot_body, 0)

        def outer(i, carry):
            for b in range(nbuf):
                kk = i * nbuf + b
                s3 = (b + 3) % nbuf
                j = kk * _NTILES + wid
                jn3 = j + 3 * _NTILES
                jp4 = j - 4 * _NTILES

                @pl.when(jn3 < nchunks)
                def _():
                    pltpu.async_copy(qpack_hbm.at[jn3], pairs_v.at[s3], sem_p)

                @pl.when((kk >= 4) & (jp4 < nchunks))
                def _():
                    pltpu.make_async_copy(
                        pbuf_v.at[b], out_hbm.at[pl.ds(0, 8)], sem_o).wait()

                @pl.when(j < nchunks)
                def _():
                    pltpu.make_async_copy(
                        h_hbm.at[pairs_v.at[b]], rows_v.at[b], sem_g).wait()
                    compute_chunk(b)
                    pltpu.async_copy(
                        pbuf_v.at[b], out_hbm.at[pl.ds(j * 8, 8)], sem_o)

                @pl.when(jn3 < nchunks)
                def _():
                    pltpu.make_async_copy(
                        qpack_hbm.at[0], pairs_v.at[s3], sem_p).wait()
                    pltpu.async_copy(
                        h_hbm.at[pairs_v.at[s3]], rows_v.at[s3], sem_g)

            return carry

        lax.fori_loop(0, iters // nbuf + 2, outer, 0)

    return k(h, qpack)


def _lane_reduce(partials, q):
    q8 = partials.shape[0]
    blk = 1000

    def red(p_ref, o_ref):
        r = lax.broadcasted_iota(jnp.int32, (128, 8), 0) // 16
        t = lax.broadcasted_iota(jnp.int32, (128, 8), 1)
        mask = (r == t).astype(jnp.float32)
        o_ref[...] = jnp.dot(p_ref[...], mask,
                             preferred_element_type=jnp.float32)

    out = pl.pallas_call(
        red,
        grid=(q8 // blk,),
        in_specs=[pl.BlockSpec((blk, 128), lambda i: (i, 0))],
        out_specs=pl.BlockSpec((blk, 8), lambda i: (i, 0)),
        out_shape=jax.ShapeDtypeStruct((q8, 8), jnp.float32),
    )(partials)
    return out.reshape(q)


def kernel(x, edge_index, edge_weight, edges, W):
    e = edge_index.shape[1]
    q = edges.shape[1]
    epack = jnp.stack(
        [edge_index[0].reshape(e // _CE, _CE),
         edge_index[1].reshape(e // _CE, _CE)], axis=1)
    ew = edge_weight.reshape(e // _CE, _CE)
    qpack = jnp.concatenate(
        [edges[0].reshape(q // _CQ, _CQ),
         edges[1].reshape(q // _CQ, _CQ)], axis=1)

    part = _segment_sum_partials(x, epack, ew)
    h = _linear(part[0], part[1], W)
    partials = _pair_partial_dots(h, qpack)
    return _lane_reduce(partials, q)
